# baseline (speedup 1.0000x reference)
import functools

import jax
import jax.numpy as jnp
from jax import lax
from jax.experimental import pallas as pl
from jax.experimental.pallas import tpu as pltpu
from jax.experimental.pallas import tpu_sc as plsc

_NC = 2
_NS = 16
_NW = _NC * _NS
_LANES = 16
_CHUNK = 112


def _sc_gather_body(table_hbm, ids_hbm, out_hbm, idx_v, rows_v, sem0, sem1):
    n_rows = ids_hbm.shape[0]
    n_table = table_hbm.shape[0]
    per_w = n_rows // _NW
    n_chunks = per_w // _CHUNK
    wid = lax.axis_index("s") * _NC + lax.axis_index("c")
    base = wid * per_w

    pltpu.sync_copy(ids_hbm.at[pl.ds(base, per_w)], idx_v)

    def _fix(i, carry):
        s = pl.ds(i * _LANES, _LANES)
        idx_v[s] = jnp.clip(idx_v[s] - 1, 0, n_table - 1)
        return carry

    lax.fori_loop(0, per_w // _LANES, _fix, 0)

    sems = (sem0, sem1)

    def _start(c, b):
        pltpu.async_copy(
            table_hbm.at[idx_v.at[pl.ds(c * _CHUNK, _CHUNK)]],
            rows_v.at[b],
            sems[b],
        )

    def _wait(c, b):
        pltpu.make_async_copy(
            table_hbm.at[idx_v.at[pl.ds(c * _CHUNK, _CHUNK)]],
            rows_v.at[b],
            sems[b],
        ).wait()

    _start(0, 0)

    def _ring(it, carry):
        c0 = it * 2
        for b in range(2):
            c = c0 + b
            nxt = c + 1

            @pl.when(nxt < n_chunks)
            def _():
                _start(nxt, (b + 1) % 2)

            _wait(c, b)
            pltpu.sync_copy(
                rows_v.at[b], out_hbm.at[pl.ds(base + c * _CHUNK, _CHUNK)]
            )
        return carry

    lax.fori_loop(0, n_chunks // 2, _ring, 0)

    if n_chunks % 2:
        c = n_chunks - 1
        _wait(c, c % 2)
        pltpu.sync_copy(
            rows_v.at[c % 2], out_hbm.at[pl.ds(base + c * _CHUNK, _CHUNK)]
        )


@functools.lru_cache(maxsize=None)
def _make_sc_gather(n_rows, n_table, dim):
    per_w = n_rows // _NW
    mesh = plsc.VectorSubcoreMesh(core_axis_name="c", subcore_axis_name="s")
    return pl.kernel(
        _sc_gather_body,
        mesh=mesh,
        out_type=jax.ShapeDtypeStruct((n_rows, dim), jnp.float32),
        scratch_types=[
            pltpu.VMEM((per_w,), jnp.int32),
            pltpu.VMEM((2, _CHUNK, dim), jnp.float32),
            pltpu.SemaphoreType.DMA,
            pltpu.SemaphoreType.DMA,
        ],
    )


def _circ_body(r2_ref, out_ref):
    dim = out_ref.shape[1]
    rows = [r2_ref[0, :, pl.ds(dim - t, dim)] for t in range(dim)]
    out_ref[...] = jnp.concatenate(rows, axis=0)[None]


def _mk_circulants(left_role, right_role):
    dim = left_role.shape[0]
    roles = jnp.stack([left_role, right_role])
    r2 = jnp.concatenate([roles, roles], axis=1).reshape(2, 1, 2 * dim)
    circ = pl.pallas_call(
        _circ_body,
        grid=(2,),
        in_specs=[pl.BlockSpec((1, 1, 2 * dim), lambda i: (i, 0, 0))],
        out_specs=pl.BlockSpec((1, dim, dim), lambda i: (i, 0, 0)),
        out_shape=jax.ShapeDtypeStruct((2, dim, dim), jnp.float32),
    )(r2)
    return circ[0], circ[1]


def _tc_bind_body(trees_ref, g_ref, ml_ref, mr_ref, out_ref):
    n_nodes = trees_ref.shape[1]
    ml = ml_ref[...].astype(jnp.bfloat16)
    mr = mr_ref[...].astype(jnp.bfloat16)
    reps = [None] * n_nodes
    for j in range(n_nodes - 1, -1, -1):
        val = g_ref[j]
        li, ri = 2 * j + 1, 2 * j + 2
        if li < n_nodes:
            val = val + jnp.dot(reps[li].astype(jnp.bfloat16), ml,
                                preferred_element_type=jnp.float32)
        if ri < n_nodes:
            val = val + jnp.dot(reps[ri].astype(jnp.bfloat16), mr,
                                preferred_element_type=jnp.float32)
        m = trees_ref[:, pl.ds(j, 1)] > 0
        reps[j] = jnp.where(m, val, 0.0)
    out_ref[...] = reps[0]


def _tc_bind(trees, gathered, ml, mr, b_blk):
    n_nodes, b, dim = gathered.shape
    return pl.pallas_call(
        _tc_bind_body,
        grid=(b // b_blk,),
        in_specs=[
            pl.BlockSpec((b_blk, n_nodes), lambda i: (i, 0)),
            pl.BlockSpec((n_nodes, b_blk, dim), lambda i: (0, i, 0)),
            pl.BlockSpec((dim, dim), lambda i: (0, 0)),
            pl.BlockSpec((dim, dim), lambda i: (0, 0)),
        ],
        out_specs=pl.BlockSpec((b_blk, dim), lambda i: (i, 0)),
        out_shape=jax.ShapeDtypeStruct((b, dim), jnp.float32),
        compiler_params=pltpu.CompilerParams(
            dimension_semantics=("arbitrary",)
        ),
    )(trees, gathered, ml, mr)


def kernel(trees, filler_weights, left_role, right_role):
    b, n_nodes = trees.shape
    n_table, dim = filler_weights.shape

    ml, mr = _mk_circulants(left_role, right_role)

    n_slabs = 4
    bs = b // n_slabs
    outs = []
    for s in range(n_slabs):
        t_s = trees[s * bs:(s + 1) * bs]
        ids = t_s.T.reshape(-1)
        g = _make_sc_gather(bs * n_nodes, n_table, dim)(filler_weights, ids)
        outs.append(_tc_bind(t_s, g.reshape(n_nodes, bs, dim), ml, mr,
                             b_blk=256))
    return jnp.concatenate(outs, axis=0)


# speedup vs baseline: 1.0463x; 1.0463x over previous
import functools

import jax
import jax.numpy as jnp
from jax import lax
from jax.experimental import pallas as pl
from jax.experimental.pallas import tpu as pltpu
from jax.experimental.pallas import tpu_sc as plsc

_NC = 2
_NS = 16
_NW = _NC * _NS
_LANES = 16
_CHUNK = 112


def _sc_gather_body(table_hbm, ids_hbm, out_hbm, idx_v, rows_v, sem0, sem1):
    n_rows = ids_hbm.shape[0]
    n_table = table_hbm.shape[0]
    per_w = n_rows // _NW
    n_chunks = per_w // _CHUNK
    wid = lax.axis_index("s") * _NC + lax.axis_index("c")
    base = wid * per_w

    pltpu.sync_copy(ids_hbm.at[pl.ds(base, per_w)], idx_v)

    def _fix(i, carry):
        s = pl.ds(i * _LANES, _LANES)
        idx_v[s] = jnp.clip(idx_v[s] - 1, 0, n_table - 1)
        return carry

    lax.fori_loop(0, per_w // _LANES, _fix, 0)

    sems = (sem0, sem1)

    def _start(c, b):
        pltpu.async_copy(
            table_hbm.at[idx_v.at[pl.ds(c * _CHUNK, _CHUNK)]],
            rows_v.at[b],
            sems[b],
        )

    def _wait(c, b):
        pltpu.make_async_copy(
            table_hbm.at[idx_v.at[pl.ds(c * _CHUNK, _CHUNK)]],
            rows_v.at[b],
            sems[b],
        ).wait()

    _start(0, 0)

    def _ring(it, carry):
        c0 = it * 2
        for b in range(2):
            c = c0 + b
            nxt = c + 1

            @pl.when(nxt < n_chunks)
            def _():
                _start(nxt, (b + 1) % 2)

            _wait(c, b)
            pltpu.sync_copy(
                rows_v.at[b], out_hbm.at[pl.ds(base + c * _CHUNK, _CHUNK)]
            )
        return carry

    lax.fori_loop(0, n_chunks // 2, _ring, 0)

    if n_chunks % 2:
        c = n_chunks - 1
        _wait(c, c % 2)
        pltpu.sync_copy(
            rows_v.at[c % 2], out_hbm.at[pl.ds(base + c * _CHUNK, _CHUNK)]
        )


@functools.lru_cache(maxsize=None)
def _make_sc_gather(n_rows, n_table, dim):
    per_w = n_rows // _NW
    mesh = plsc.VectorSubcoreMesh(core_axis_name="c", subcore_axis_name="s")
    return pl.kernel(
        _sc_gather_body,
        mesh=mesh,
        out_type=jax.ShapeDtypeStruct((n_rows, dim), jnp.float32),
        scratch_types=[
            pltpu.VMEM((per_w,), jnp.int32),
            pltpu.VMEM((2, _CHUNK, dim), jnp.float32),
            pltpu.SemaphoreType.DMA,
            pltpu.SemaphoreType.DMA,
        ],
    )


def _circ_body(r2_ref, out_ref):
    dim = out_ref.shape[1]
    rows = [r2_ref[0, :, pl.ds(dim - t, dim)] for t in range(dim)]
    out_ref[...] = jnp.concatenate(rows, axis=0)[None]


def _mk_circulants(left_role, right_role):
    dim = left_role.shape[0]
    roles = jnp.stack([left_role, right_role])
    r2 = jnp.concatenate([roles, roles], axis=1).reshape(2, 1, 2 * dim)
    circ = pl.pallas_call(
        _circ_body,
        grid=(2,),
        in_specs=[pl.BlockSpec((1, 1, 2 * dim), lambda i: (i, 0, 0))],
        out_specs=pl.BlockSpec((1, dim, dim), lambda i: (i, 0, 0)),
        out_shape=jax.ShapeDtypeStruct((2, dim, dim), jnp.float32),
    )(r2)
    return circ[0], circ[1]


def _tc_bind_body(trees_ref, g_ref, ml_ref, mr_ref, out_ref):
    n_nodes = trees_ref.shape[1]
    ml = ml_ref[...].astype(jnp.bfloat16)
    mr = mr_ref[...].astype(jnp.bfloat16)
    reps = [None] * n_nodes
    for j in range(n_nodes - 1, -1, -1):
        val = g_ref[j]
        li, ri = 2 * j + 1, 2 * j + 2
        if li < n_nodes:
            val = val + jnp.dot(reps[li].astype(jnp.bfloat16), ml,
                                preferred_element_type=jnp.float32)
        if ri < n_nodes:
            val = val + jnp.dot(reps[ri].astype(jnp.bfloat16), mr,
                                preferred_element_type=jnp.float32)
        m = trees_ref[:, pl.ds(j, 1)] > 0
        reps[j] = jnp.where(m, val, 0.0)
    out_ref[...] = reps[0]


def _tc_bind(trees, gathered, ml, mr, b_blk):
    n_nodes, b, dim = gathered.shape
    return pl.pallas_call(
        _tc_bind_body,
        grid=(b // b_blk,),
        in_specs=[
            pl.BlockSpec((b_blk, n_nodes), lambda i: (i, 0)),
            pl.BlockSpec((n_nodes, b_blk, dim), lambda i: (0, i, 0)),
            pl.BlockSpec((dim, dim), lambda i: (0, 0)),
            pl.BlockSpec((dim, dim), lambda i: (0, 0)),
        ],
        out_specs=pl.BlockSpec((b_blk, dim), lambda i: (i, 0)),
        out_shape=jax.ShapeDtypeStruct((b, dim), jnp.float32),
        compiler_params=pltpu.CompilerParams(
            dimension_semantics=("arbitrary",)
        ),
    )(trees, gathered, ml, mr)


def kernel(trees, filler_weights, left_role, right_role):
    b, n_nodes = trees.shape
    n_table, dim = filler_weights.shape

    ml, mr = _mk_circulants(left_role, right_role)

    n_slabs = 1
    bs = b // n_slabs
    outs = []
    for s in range(n_slabs):
        t_s = trees[s * bs:(s + 1) * bs]
        ids = t_s.T.reshape(-1)
        g = _make_sc_gather(bs * n_nodes, n_table, dim)(filler_weights, ids)
        outs.append(_tc_bind(t_s, g.reshape(n_nodes, bs, dim), ml, mr,
                             b_blk=256))
    return jnp.concatenate(outs, axis=0)
